# baseline (speedup 1.0000x reference)
import jax
import jax.numpy as jnp
from jax.experimental import pallas as pl


def _broadcast_body(pe_ref, o_ref):
    o_ref[...] = jnp.broadcast_to(pe_ref[...][None], o_ref.shape)


def kernel(x, pos_embed):
    batch = x.shape[0]
    max_len, d_model = pos_embed.shape
    bc = 128
    return pl.pallas_call(
        _broadcast_body,
        grid=(batch // bc,),
        in_specs=[pl.BlockSpec((max_len, d_model), lambda i: (0, 0))],
        out_specs=pl.BlockSpec((bc, max_len, d_model), lambda i: (i, 0, 0)),
        out_shape=jax.ShapeDtypeStruct((batch, max_len, d_model), jnp.float32),
    )(pos_embed)


# speedup vs baseline: 1.6538x; 1.6538x over previous
import jax
import jax.numpy as jnp
from jax.experimental import pallas as pl


def _broadcast_body(pe_ref, o_ref):
    o_ref[...] = jnp.broadcast_to(pe_ref[...], o_ref.shape)


def kernel(x, pos_embed):
    batch = x.shape[0]
    max_len, d_model = pos_embed.shape
    row = max_len * d_model
    pe_flat = pos_embed.reshape(1, row)
    bc = 128
    out = pl.pallas_call(
        _broadcast_body,
        grid=(batch // bc,),
        in_specs=[pl.BlockSpec((1, row), lambda i: (0, 0))],
        out_specs=pl.BlockSpec((bc, row), lambda i: (i, 0)),
        out_shape=jax.ShapeDtypeStruct((batch, row), jnp.float32),
    )(pe_flat)
    return out.reshape(batch, max_len, d_model)
